# baseline (speedup 1.0000x reference)
import functools

import jax
import jax.numpy as jnp
from jax import lax
from jax.experimental import pallas as pl
from jax.experimental.pallas import tpu as pltpu
from jax.experimental.pallas import tpu_sc as plsc

_B, _H, _W, _D = 2, 224, 224, 256
_NC, _NS = 2, 16
_NW = _NC * _NS
_HPW = _H // _NW
_Q = 4
_QROWS = _W // _Q
_L = 16

_mesh = plsc.VectorSubcoreMesh(
    core_axis_name="c", subcore_axis_name="s", num_cores=_NC, num_subcores=_NS
)


@functools.partial(
    pl.kernel,
    mesh=_mesh,
    out_type=jax.ShapeDtypeStruct((_B, _H * _W, _D), jnp.float32),
    scratch_types=[
        pltpu.VMEM((_W, _D), jnp.float32),
        pltpu.VMEM((16, _D), jnp.float32),
        pltpu.VMEM((_Q, _QROWS, _D), jnp.float32),
        pltpu.SemaphoreType.DMA,
        [pltpu.SemaphoreType.DMA] * _Q,
    ],
)
def _sc_pos_enc(
    row_hbm, col_hbm, out_hbm, col_buf, row_buf, obuf, col_sem, osems
):
    wid = lax.axis_index("s") * _NC + lax.axis_index("c")
    h0 = wid * _HPW

    base8 = (h0 // 8) * 8
    roff = h0 - base8
    col_loads = [
        pltpu.async_copy(
            col_hbm.at[pl.ds(q * _QROWS, _QROWS)],
            obuf_col.at[pl.ds(q * _QROWS, _QROWS)],
            col_sem,
        )
        for q, obuf_col in ((q, col_buf) for q in range(_Q))
    ]
    pltpu.sync_copy(row_hbm.at[pl.ds(base8, 16)], row_buf)

    pending = [None] * _Q
    for hl in range(_HPW):
        row_vecs = [
            row_buf[roff + hl, pl.ds(j * _L, _L)] for j in range(_D // _L)
        ]
        for q in range(_Q):
            if hl == 0:
                col_loads[q].wait()
            if pending[q] is not None:
                for c in pending[q]:
                    c.wait()

            @plsc.parallel_loop(0, _QROWS, unroll=8)
            def _(w, q=q, row_vecs=row_vecs):
                for j in range(_D // _L):
                    obuf[q, w, pl.ds(j * _L, _L)] = (
                        col_buf[q * _QROWS + w, pl.ds(j * _L, _L)]
                        + row_vecs[j]
                    )

            base = (h0 + hl) * _W + q * _QROWS
            c0 = pltpu.async_copy(
                obuf.at[q], out_hbm.at[0, pl.ds(base, _QROWS)], osems[q]
            )
            c1 = pltpu.async_copy(
                obuf.at[q], out_hbm.at[1, pl.ds(base, _QROWS)], osems[q]
            )
            pending[q] = (c0, c1)

    for q in range(_Q):
        for c in pending[q]:
            c.wait()


def kernel(batch_size, height, width, row_embed, col_embed):
    return _sc_pos_enc(row_embed, col_embed)


# speedup vs baseline: 1.1564x; 1.1564x over previous
import functools

import jax
import jax.numpy as jnp
from jax import lax
from jax.experimental import pallas as pl
from jax.experimental.pallas import tpu as pltpu
from jax.experimental.pallas import tpu_sc as plsc

_B, _H, _W, _D = 2, 224, 224, 256
_NC, _NS = 2, 16
_NW = _NC * _NS
_HPW = 2 * _H // _NW
_HALF = _W // 2
_NSLOT = 3
_RWIN = 24
_L = 16

_mesh = plsc.VectorSubcoreMesh(
    core_axis_name="c", subcore_axis_name="s", num_cores=_NC, num_subcores=_NS
)


@functools.partial(
    pl.kernel,
    mesh=_mesh,
    out_type=jax.ShapeDtypeStruct((_B, _H * _W, _D), jnp.float32),
    scratch_types=[
        pltpu.VMEM((_HALF, _D), jnp.float32),
        pltpu.VMEM((_RWIN, _D), jnp.float32),
        pltpu.VMEM((_NSLOT, _HALF, _D), jnp.float32),
        [pltpu.SemaphoreType.DMA] * _NSLOT,
    ],
)
def _sc_pos_enc(row_hbm, col_hbm, out_hbm, col_buf, row_buf, obuf, osems):
    wid = lax.axis_index("s") * _NC + lax.axis_index("c")
    g = wid % 2
    h0 = (wid // 2) * _HPW

    base8 = (h0 // 8) * 8
    roff = h0 - base8
    pltpu.sync_copy(col_hbm.at[pl.ds(g * _HALF, _HALF)], col_buf)
    pltpu.sync_copy(row_hbm.at[pl.ds(base8, _RWIN)], row_buf)

    pending = [None] * _NSLOT
    for hl in range(_HPW):
        row_vecs = [
            row_buf[roff + hl, pl.ds(j * _L, _L)] for j in range(_D // _L)
        ]
        s = hl % _NSLOT
        if pending[s] is not None:
            for c in pending[s]:
                c.wait()

        @plsc.parallel_loop(0, _HALF, unroll=8)
        def _(w, s=s, row_vecs=row_vecs):
            for j in range(_D // _L):
                obuf[s, w, pl.ds(j * _L, _L)] = (
                    col_buf[w, pl.ds(j * _L, _L)] + row_vecs[j]
                )

        base = (h0 + hl) * _W + g * _HALF
        c0 = pltpu.async_copy(
            obuf.at[s], out_hbm.at[0, pl.ds(base, _HALF)], osems[s]
        )
        c1 = pltpu.async_copy(
            obuf.at[s], out_hbm.at[1, pl.ds(base, _HALF)], osems[s]
        )
        pending[s] = (c0, c1)

    for s in range(_NSLOT):
        for c in pending[s]:
            c.wait()


def kernel(batch_size, height, width, row_embed, col_embed):
    return _sc_pos_enc(row_embed, col_embed)
